# baseline (device time: 52212 ns/iter reference)
import jax
import jax.numpy as jnp
from jax import lax
from jax.experimental import pallas as pl
from jax.experimental.pallas import tpu as pltpu

N_DEV = 4
N_TOK = 2048
D = 512
H = 1024
H2 = H // 2
E_LOCAL = 8
E_TOTAL = 32
CHUNK = N_TOK // N_DEV


def kernel(x, router_W, route_idx, expert_W):
    def body(x_ref, rw_ref, idx_ref, ew_ref, out_ref,
             ewb_ref, w_ref, xb_ref, commR_ref, commL_ref,
             sendR, recvR, sendL, recvL):
        my_i = lax.axis_index("i")
        left = lax.rem(my_i - 1 + N_DEV, N_DEV)
        right = lax.rem(my_i + 1, N_DEV)

        barrier_sem = pltpu.get_barrier_semaphore()
        pl.semaphore_signal(barrier_sem, inc=1, device_id=(left,),
                            device_id_type=pl.DeviceIdType.MESH)
        pl.semaphore_signal(barrier_sem, inc=1, device_id=(right,),
                            device_id_type=pl.DeviceIdType.MESH)
        pl.semaphore_wait(barrier_sem, 2)

        xf = x_ref[:, :]
        scores = jnp.dot(xf, rw_ref[:, :], preferred_element_type=jnp.float32)
        m = jnp.max(scores, axis=-1, keepdims=True)
        p = jnp.exp(scores - m)
        p = p / jnp.sum(p, axis=-1, keepdims=True)
        iota = lax.broadcasted_iota(jnp.int32, (N_TOK, E_TOTAL), 1)
        oh0 = iota == idx_ref[:, 0:1]
        oh1 = iota == idx_ref[:, 1:2]
        p0 = jnp.sum(jnp.where(oh0, p, 0.0), axis=-1, keepdims=True)
        p1 = jnp.sum(jnp.where(oh1, p, 0.0), axis=-1, keepdims=True)
        w_ref[:, :] = jnp.where(oh0 | oh1, p, 0.0) / (p0 + p1)

        xb_ref[:, :] = xf.astype(jnp.bfloat16)
        for j in range(E_LOCAL):
            ewb_ref[j, :, :] = ew_ref[j, :, :].astype(jnp.bfloat16)

        iota_c = lax.broadcasted_iota(jnp.int32, (CHUNK, E_TOTAL), 1)

        def chunk_partial(c):
            row0 = c * CHUNK
            wc = w_ref[pl.ds(row0, CHUNK), :]
            xc = xb_ref[pl.ds(row0, CHUNK), :]
            acc = jnp.zeros((CHUNK, H), jnp.float32)
            for j in range(E_LOCAL):
                ge = my_i * E_LOCAL + j
                col = jnp.sum(jnp.where(iota_c == ge, wc, 0.0), axis=-1,
                              keepdims=True)
                yj = jnp.dot(xc, ewb_ref[j, :, :],
                             preferred_element_type=jnp.float32)
                acc = acc + col * yj
            return acc

        def hop(comm_ref, ssems, rsems, h, tgt):
            src_slot = 3 if h == 0 else h - 1
            return pltpu.make_async_remote_copy(
                src_ref=comm_ref.at[src_slot],
                dst_ref=comm_ref.at[h],
                send_sem=ssems.at[h],
                recv_sem=rsems.at[h],
                device_id=(tgt,),
                device_id_type=pl.DeviceIdType.MESH,
            )

        A = chunk_partial(lax.rem(my_i + 3, N_DEV))
        commR_ref[3, :, :] = A[:, :H2].astype(jnp.bfloat16)
        rR0 = hop(commR_ref, sendR, recvR, 0, right)
        rR0.start()

        B = chunk_partial(lax.rem(my_i + 1, N_DEV))
        commL_ref[3, :, :] = B[:, H2:].astype(jnp.bfloat16)
        rL0 = hop(commL_ref, sendL, recvL, 0, left)
        rL0.start()

        C = chunk_partial(lax.rem(my_i + 2, N_DEV))

        rR0.wait_recv()
        commR_ref[0, :, :] = (
            commR_ref[0, :, :].astype(jnp.float32) + C[:, :H2]
        ).astype(jnp.bfloat16)
        rR1 = hop(commR_ref, sendR, recvR, 1, right)
        rR1.start()

        rL0.wait_recv()
        commL_ref[0, :, :] = (
            commL_ref[0, :, :].astype(jnp.float32) + C[:, H2:]
        ).astype(jnp.bfloat16)
        rL1 = hop(commL_ref, sendL, recvL, 1, left)
        rL1.start()

        Dn = chunk_partial(my_i)

        rR1.wait_recv()
        commR_ref[1, :, :] = (
            commR_ref[1, :, :].astype(jnp.float32) + B[:, :H2]
        ).astype(jnp.bfloat16)
        rR2 = hop(commR_ref, sendR, recvR, 2, right)
        rR2.start()

        rL1.wait_recv()
        commL_ref[1, :, :] = (
            commL_ref[1, :, :].astype(jnp.float32) + A[:, H2:]
        ).astype(jnp.bfloat16)
        rL2 = hop(commL_ref, sendL, recvL, 2, left)
        rL2.start()

        rR2.wait_recv()
        out_ref[:, :H2] = commR_ref[2, :, :].astype(jnp.float32) + Dn[:, :H2]
        rL2.wait_recv()
        out_ref[:, H2:] = commL_ref[2, :, :].astype(jnp.float32) + Dn[:, H2:]

        for r in (rR0, rR1, rR2, rL0, rL1, rL2):
            r.wait_send()

    return pl.pallas_call(
        body,
        out_shape=jax.ShapeDtypeStruct((CHUNK, H), jnp.float32),
        in_specs=[
            pl.BlockSpec(memory_space=pltpu.VMEM),
            pl.BlockSpec(memory_space=pltpu.VMEM),
            pl.BlockSpec(memory_space=pltpu.VMEM),
            pl.BlockSpec(memory_space=pltpu.VMEM),
        ],
        out_specs=pl.BlockSpec(memory_space=pltpu.VMEM),
        scratch_shapes=[
            pltpu.VMEM((E_LOCAL, D, H), jnp.bfloat16),
            pltpu.VMEM((N_TOK, E_TOTAL), jnp.float32),
            pltpu.VMEM((N_TOK, D), jnp.bfloat16),
            pltpu.VMEM((4, CHUNK, H2), jnp.bfloat16),
            pltpu.VMEM((4, CHUNK, H2), jnp.bfloat16),
            pltpu.SemaphoreType.DMA((N_DEV - 1,)),
            pltpu.SemaphoreType.DMA((N_DEV - 1,)),
            pltpu.SemaphoreType.DMA((N_DEV - 1,)),
            pltpu.SemaphoreType.DMA((N_DEV - 1,)),
        ],
        compiler_params=pltpu.CompilerParams(
            collective_id=0, vmem_limit_bytes=100 * 1024 * 1024
        ),
    )(x, router_W, route_idx, expert_W)


# device time: 48561 ns/iter; 1.0752x vs baseline; 1.0752x over previous
import jax
import jax.numpy as jnp
from jax import lax
from jax.experimental import pallas as pl
from jax.experimental.pallas import tpu as pltpu

N_DEV = 4
N_TOK = 2048
D = 512
H = 1024
H2 = H // 2
E_LOCAL = 8
E_TOTAL = 32
CHUNK = N_TOK // N_DEV


def kernel(x, router_W, route_idx, expert_W):
    def body(x_ref, rw_ref, idx_ref, ew_ref, out_ref,
             ewb_ref, w_ref, xb_ref, commR_ref, commL_ref,
             sendR, recvR, sendL, recvL):
        my_i = lax.axis_index("i")
        left = lax.rem(my_i - 1 + N_DEV, N_DEV)
        right = lax.rem(my_i + 1, N_DEV)

        barrier_sem = pltpu.get_barrier_semaphore()
        pl.semaphore_signal(barrier_sem, inc=1, device_id=(left,),
                            device_id_type=pl.DeviceIdType.MESH)
        pl.semaphore_signal(barrier_sem, inc=1, device_id=(right,),
                            device_id_type=pl.DeviceIdType.MESH)
        pl.semaphore_wait(barrier_sem, 2)

        xf = x_ref[:, :]
        scores = jnp.dot(xf, rw_ref[:, :], preferred_element_type=jnp.float32)
        m = jnp.max(scores, axis=-1, keepdims=True)
        p = jnp.exp(scores - m)
        p = p / jnp.sum(p, axis=-1, keepdims=True)
        iota = lax.broadcasted_iota(jnp.int32, (N_TOK, E_TOTAL), 1)
        oh0 = iota == idx_ref[:, 0:1]
        oh1 = iota == idx_ref[:, 1:2]
        p0 = jnp.sum(jnp.where(oh0, p, 0.0), axis=-1, keepdims=True)
        p1 = jnp.sum(jnp.where(oh1, p, 0.0), axis=-1, keepdims=True)
        w_ref[:, :] = jnp.where(oh0 | oh1, p, 0.0) / (p0 + p1)

        xb_ref[:, :] = xf.astype(jnp.bfloat16)
        for j in range(E_LOCAL):
            ewb_ref[j, :, :] = ew_ref[j, :, :].astype(jnp.bfloat16)

        iota_c = lax.broadcasted_iota(jnp.int32, (CHUNK, E_TOTAL), 1)

        def chunk_partial(c, lo):
            row0 = c * CHUNK
            wc = w_ref[pl.ds(row0, CHUNK), :]
            xc = xb_ref[pl.ds(row0, CHUNK), :]
            acc = jnp.zeros((CHUNK, H2), jnp.float32)
            for j in range(E_LOCAL):
                ge = my_i * E_LOCAL + j
                col = jnp.sum(jnp.where(iota_c == ge, wc, 0.0), axis=-1,
                              keepdims=True)
                yj = jnp.dot(xc, ewb_ref[j, :, pl.ds(lo, H2)],
                             preferred_element_type=jnp.float32)
                acc = acc + col * yj
            return acc

        def hop(comm_ref, ssems, rsems, h, tgt):
            src_slot = 3 if h == 0 else h - 1
            return pltpu.make_async_remote_copy(
                src_ref=comm_ref.at[src_slot],
                dst_ref=comm_ref.at[h],
                send_sem=ssems.at[h],
                recv_sem=rsems.at[h],
                device_id=(tgt,),
                device_id_type=pl.DeviceIdType.MESH,
            )

        cA = lax.rem(my_i + 3, N_DEV)
        cB = lax.rem(my_i + 1, N_DEV)
        cC = lax.rem(my_i + 2, N_DEV)

        commR_ref[3, :, :] = chunk_partial(cA, 0).astype(jnp.bfloat16)
        rR0 = hop(commR_ref, sendR, recvR, 0, right)
        rR0.start()

        commL_ref[3, :, :] = chunk_partial(cB, H2).astype(jnp.bfloat16)
        rL0 = hop(commL_ref, sendL, recvL, 0, left)
        rL0.start()

        C_R = chunk_partial(cC, 0)
        rR0.wait_recv()
        commR_ref[0, :, :] = (
            commR_ref[0, :, :].astype(jnp.float32) + C_R
        ).astype(jnp.bfloat16)
        rR1 = hop(commR_ref, sendR, recvR, 1, right)
        rR1.start()

        C_L = chunk_partial(cC, H2)
        rL0.wait_recv()
        commL_ref[0, :, :] = (
            commL_ref[0, :, :].astype(jnp.float32) + C_L
        ).astype(jnp.bfloat16)
        rL1 = hop(commL_ref, sendL, recvL, 1, left)
        rL1.start()

        B_R = chunk_partial(cB, 0)
        rR1.wait_recv()
        commR_ref[1, :, :] = (
            commR_ref[1, :, :].astype(jnp.float32) + B_R
        ).astype(jnp.bfloat16)
        rR2 = hop(commR_ref, sendR, recvR, 2, right)
        rR2.start()

        A_L = chunk_partial(cA, H2)
        rL1.wait_recv()
        commL_ref[1, :, :] = (
            commL_ref[1, :, :].astype(jnp.float32) + A_L
        ).astype(jnp.bfloat16)
        rL2 = hop(commL_ref, sendL, recvL, 2, left)
        rL2.start()

        D_R = chunk_partial(my_i, 0)
        rR2.wait_recv()
        out_ref[:, :H2] = commR_ref[2, :, :].astype(jnp.float32) + D_R

        D_L = chunk_partial(my_i, H2)
        rL2.wait_recv()
        out_ref[:, H2:] = commL_ref[2, :, :].astype(jnp.float32) + D_L

        for r in (rR0, rR1, rR2, rL0, rL1, rL2):
            r.wait_send()

    return pl.pallas_call(
        body,
        out_shape=jax.ShapeDtypeStruct((CHUNK, H), jnp.float32),
        in_specs=[
            pl.BlockSpec(memory_space=pltpu.VMEM),
            pl.BlockSpec(memory_space=pltpu.VMEM),
            pl.BlockSpec(memory_space=pltpu.VMEM),
            pl.BlockSpec(memory_space=pltpu.VMEM),
        ],
        out_specs=pl.BlockSpec(memory_space=pltpu.VMEM),
        scratch_shapes=[
            pltpu.VMEM((E_LOCAL, D, H), jnp.bfloat16),
            pltpu.VMEM((N_TOK, E_TOTAL), jnp.float32),
            pltpu.VMEM((N_TOK, D), jnp.bfloat16),
            pltpu.VMEM((4, CHUNK, H2), jnp.bfloat16),
            pltpu.VMEM((4, CHUNK, H2), jnp.bfloat16),
            pltpu.SemaphoreType.DMA((N_DEV - 1,)),
            pltpu.SemaphoreType.DMA((N_DEV - 1,)),
            pltpu.SemaphoreType.DMA((N_DEV - 1,)),
            pltpu.SemaphoreType.DMA((N_DEV - 1,)),
        ],
        compiler_params=pltpu.CompilerParams(
            collective_id=0, vmem_limit_bytes=100 * 1024 * 1024
        ),
    )(x, router_W, route_idx, expert_W)
